# baseline (speedup 1.0000x reference)
import functools

import jax
import jax.numpy as jnp
from jax import lax
from jax.experimental import pallas as pl
from jax.experimental.pallas import tpu as pltpu
from jax.experimental.pallas import tpu_sc as plsc

_EPS = 1e-12

_N = 16384
_C = 32
_M = 8192
_T = 512
_G = _N // _T

_W = 4096
_NWIN = _M // _W


def _argmax_body(x_ref, e_ref, idx_ref, table_ref, en_ref):
    @pl.when(pl.program_id(0) == 0)
    def _():
        emb = e_ref[...]
        en_c = emb / jnp.maximum(
            jnp.sqrt(jnp.sum(emb * emb, axis=0, keepdims=True)), _EPS)
        en_ref[...] = en_c
        table_ref[...] = en_c.T

    en = en_ref[...]
    x = x_ref[...]
    xn = x / jnp.maximum(
        jnp.sqrt(jnp.sum(x * x, axis=-1, keepdims=True)), _EPS)

    sims = jax.lax.dot_general(
        xn, en, (((1,), (0,)), ((), ())),
        preferred_element_type=jnp.float32,
    )

    lane = jax.lax.broadcasted_iota(jnp.int32, (_T, 128), 1)
    acc_v = None
    for w in range(_NWIN):
        s3 = sims[:, w * _W:(w + 1) * _W].reshape(_T, _W // 128, 128)
        vmaxl = jnp.max(s3, axis=1)
        kfirst = jnp.argmax(s3, axis=1).astype(jnp.int32)
        vmax = jnp.max(vmaxl, axis=1)
        cand = jnp.where(vmaxl == vmax[:, None],
                         kfirst * 128 + lane, jnp.int32(2**30))
        varg = jnp.min(cand, axis=1) + w * _W
        vmax_r = vmax.astype(jnp.bfloat16).astype(jnp.float32)
        if acc_v is None:
            acc_v, acc_i = vmax_r, varg
        else:
            steal = vmax > acc_v
            acc_v = jnp.where(steal, vmax_r, acc_v)
            acc_i = jnp.where(steal, varg, acc_i)
    idx_ref[0, 0, :] = acc_i


def _tc_argmax(xn, en):
    idx3, table = pl.pallas_call(
        _argmax_body,
        grid=(_G,),
        in_specs=[
            pl.BlockSpec((_T, _C), lambda i: (i, 0)),
            pl.BlockSpec((_C, _M), lambda i: (0, 0)),
        ],
        out_specs=[
            pl.BlockSpec((1, 1, _T), lambda i: (i, 0, 0)),
            pl.BlockSpec((_M, _C), lambda i: (0, 0)),
        ],
        out_shape=[
            jax.ShapeDtypeStruct((_G, 1, _T), jnp.int32),
            jax.ShapeDtypeStruct((_M, _C), jnp.float32),
        ],
        scratch_shapes=[pltpu.VMEM((_C, _M), jnp.float32)],
    )(xn, en)
    return idx3.reshape(_N), table


_NC = 2
_NS = 16
_NW = _NC * _NS
_BPW = _N // _NW


def _gather_body(table_hbm, idx_hbm, out_hbm, idx_v, rows_v, sem):
    wid = lax.axis_index("s") * _NC + lax.axis_index("c")
    base = wid * _BPW
    pltpu.sync_copy(idx_hbm.at[pl.ds(base, _BPW)], idx_v)
    pltpu.async_copy(table_hbm.at[idx_v], rows_v, sem).wait()
    pltpu.sync_copy(rows_v, out_hbm.at[pl.ds(base, _BPW)])


@functools.cache
def _sc_gather():
    return pl.kernel(
        _gather_body,
        out_type=jax.ShapeDtypeStruct((_N, _C), jnp.float32),
        mesh=plsc.VectorSubcoreMesh(core_axis_name="c", subcore_axis_name="s"),
        scratch_types=[
            pltpu.VMEM((_BPW,), jnp.int32),
            pltpu.VMEM((_BPW, _C), jnp.float32),
            pltpu.SemaphoreType.DMA,
        ],
        compiler_params=pltpu.CompilerParams(use_tc_tiling_on_sc=False),
    )


def kernel(input, embeddings):
    flat_x = input.reshape(-1, input.shape[-1])
    idx, table = _tc_argmax(flat_x, embeddings)
    quantized = _sc_gather()(table, idx)
    return quantized.reshape(*input.shape[:-1], _C)


# speedup vs baseline: 2.4325x; 2.4325x over previous
import functools

import jax
import jax.numpy as jnp
from jax import lax
from jax.experimental import pallas as pl
from jax.experimental.pallas import tpu as pltpu
from jax.experimental.pallas import tpu_sc as plsc

_EPS = 1e-12

_N = 16384
_C = 32
_M = 8192
_T = 512
_G = _N // _T

_W = 4096
_NWIN = _M // _W


def _argmax_body(x_ref, e_ref, idx_ref, table_ref, en_ref):
    @pl.when(pl.program_id(0) == 0)
    def _():
        emb = e_ref[...]
        en_c = emb / jnp.maximum(
            jnp.sqrt(jnp.sum(emb * emb, axis=0, keepdims=True)), _EPS)
        en_ref[...] = en_c
        table_ref[...] = en_c.T

    en = en_ref[...]
    x = x_ref[...]
    xn = x / jnp.maximum(
        jnp.sqrt(jnp.sum(x * x, axis=-1, keepdims=True)), _EPS)

    sims = jax.lax.dot_general(
        xn, en, (((1,), (0,)), ((), ())),
        preferred_element_type=jnp.float32,
    )

    acc_v = None
    for w in range(_NWIN):
        blk = sims[:, w * _W:(w + 1) * _W]
        vmax = jnp.max(blk, axis=1)
        varg = jnp.argmax(blk, axis=1).astype(jnp.int32) + w * _W
        vmax_r = vmax.astype(jnp.bfloat16).astype(jnp.float32)
        if acc_v is None:
            acc_v, acc_i = vmax_r, varg
        else:
            steal = vmax > acc_v
            acc_v = jnp.where(steal, vmax_r, acc_v)
            acc_i = jnp.where(steal, varg, acc_i)
    idx_ref[0, 0, :] = acc_i


def _tc_argmax(xn, en):
    idx3, table = pl.pallas_call(
        _argmax_body,
        grid=(_G,),
        in_specs=[
            pl.BlockSpec((_T, _C), lambda i: (i, 0)),
            pl.BlockSpec((_C, _M), lambda i: (0, 0)),
        ],
        out_specs=[
            pl.BlockSpec((1, 1, _T), lambda i: (i, 0, 0)),
            pl.BlockSpec((_M, _C), lambda i: (0, 0)),
        ],
        out_shape=[
            jax.ShapeDtypeStruct((_G, 1, _T), jnp.int32),
            jax.ShapeDtypeStruct((_M, _C), jnp.float32),
        ],
        scratch_shapes=[pltpu.VMEM((_C, _M), jnp.float32)],
    )(xn, en)
    return idx3.reshape(_N), table


_NC = 2
_NS = 16
_NW = _NC * _NS
_BPW = _N // _NW


def _gather_body(table_hbm, idx_hbm, out_hbm, idx_v, rows_v, sem):
    wid = lax.axis_index("s") * _NC + lax.axis_index("c")
    base = wid * _BPW
    pltpu.sync_copy(idx_hbm.at[pl.ds(base, _BPW)], idx_v)
    pltpu.async_copy(table_hbm.at[idx_v], rows_v, sem).wait()
    pltpu.sync_copy(rows_v, out_hbm.at[pl.ds(base, _BPW)])


@functools.cache
def _sc_gather():
    return pl.kernel(
        _gather_body,
        out_type=jax.ShapeDtypeStruct((_N, _C), jnp.float32),
        mesh=plsc.VectorSubcoreMesh(core_axis_name="c", subcore_axis_name="s"),
        scratch_types=[
            pltpu.VMEM((_BPW,), jnp.int32),
            pltpu.VMEM((_BPW, _C), jnp.float32),
            pltpu.SemaphoreType.DMA,
        ],
        compiler_params=pltpu.CompilerParams(use_tc_tiling_on_sc=False),
    )


def kernel(input, embeddings):
    flat_x = input.reshape(-1, input.shape[-1])
    idx, table = _tc_argmax(flat_x, embeddings)
    quantized = _sc_gather()(table, idx)
    return quantized.reshape(*input.shape[:-1], _C)
